# baseline (speedup 1.0000x reference)
import functools

import jax
import jax.numpy as jnp
from jax import lax
from jax.experimental import pallas as pl
from jax.experimental.pallas import tpu as pltpu
from jax.experimental.pallas import tpu_sc as plsc

N = 100000
D = 128
K = N // 4
ROWS_PER_BLK = 32768
NBLK = (N + ROWS_PER_BLK - 1) // ROWS_PER_BLK
N_PAD = NBLK * ROWS_PER_BLK

SR, SC_ = 256, 128
M = SR * SC_

BIN_SHIFT = 19
NBINS = 2048

NW = 32
CHUNK = 3136
CHUNK_LAST = N - 31 * CHUNK
NVREG = CHUNK // 16
REGION = M // NW

GROWS = K // NW + 3

_MESH = plsc.VectorSubcoreMesh(core_axis_name="c", subcore_axis_name="s")


def _iota16():
    return lax.iota(jnp.int32, 16)


def _score_body(x_ref, w_ref, b_ref, s_ref):
    zt = lax.dot_general(w_ref[...], x_ref[...],
                         (((1,), (1,)), ((), ())))
    s_ref[...] = jax.nn.sigmoid(zt + b_ref[0, 0])[0, :]


def _scores(x, W, b):
    return pl.pallas_call(
        _score_body,
        grid=(NBLK,),
        in_specs=[
            pl.BlockSpec((ROWS_PER_BLK, D), lambda i: (i, 0)),
            pl.BlockSpec((1, D), lambda i: (0, 0)),
            pl.BlockSpec((1, 1), lambda i: (0, 0)),
        ],
        out_specs=pl.BlockSpec((ROWS_PER_BLK,), lambda i: (i,)),
        out_shape=jax.ShapeDtypeStruct((N_PAD,), jnp.float32),
    )(x, W.reshape(1, D), b.reshape(1, 1))


@functools.partial(
    pl.kernel,
    mesh=_MESH,
    compiler_params=pltpu.CompilerParams(needs_layout_passes=False),
    out_type=jax.ShapeDtypeStruct((NW, NBINS), jnp.int32),
    scratch_types=[
        pltpu.VMEM((CHUNK,), jnp.float32),
        pltpu.VMEM((NBINS * 16,), jnp.int32),
        pltpu.VMEM((NBINS,), jnp.int32),
    ],
)
def _k2a(score_hbm, hists_hbm, chunk, hlanes, hloc):
    wid = lax.axis_index("s") * 2 + lax.axis_index("c")
    base = wid * CHUNK
    iota = _iota16()
    zeros = jnp.zeros((16,), jnp.int32)
    ones = jnp.ones((16,), jnp.int32)

    def zbody(i, _):
        hlanes[pl.ds(i * 16, 16)] = zeros
        return 0
    lax.fori_loop(0, NBINS, zbody, 0)

    @pl.when(wid < NW - 1)
    def _():
        pltpu.sync_copy(score_hbm.at[pl.ds(base, CHUNK)], chunk)

    @pl.when(wid == NW - 1)
    def _():
        pltpu.sync_copy(score_hbm.at[pl.ds(base, CHUNK_LAST)],
                        chunk.at[pl.ds(0, CHUNK_LAST)])

    def hbody(i, _):
        key = lax.bitcast_convert_type(chunk[pl.ds(i * 16, 16)], jnp.int32)
        b_ = lax.shift_right_logical(key, 19)
        valid = (base + i * 16 + iota) < N
        plsc.addupdate_scatter(hlanes, [b_ * 16 + iota], ones, mask=valid)
        return 0
    lax.fori_loop(0, NVREG, hbody, 0)

    def fbody(v, _):
        acc = jnp.zeros((16,), jnp.int32)
        for l in range(16):
            acc = acc + plsc.load_gather(hlanes, [v * 256 + iota * 16 + l])
        hloc[pl.ds(v * 16, 16)] = acc
        return 0
    lax.fori_loop(0, NBINS // 16, fbody, 0)

    pltpu.sync_copy(hloc, hists_hbm.at[wid])


@functools.partial(
    pl.kernel,
    mesh=_MESH,
    compiler_params=pltpu.CompilerParams(needs_layout_passes=False),
    out_type=(jax.ShapeDtypeStruct((M,), jnp.int32),
              jax.ShapeDtypeStruct((M,), jnp.int32)),
    scratch_types=[
        pltpu.VMEM((NW, NBINS), jnp.int32),
        pltpu.VMEM((NBINS,), jnp.int32),
        pltpu.VMEM((CHUNK,), jnp.float32),
        pltpu.VMEM((REGION,), jnp.int32),
        pltpu.VMEM((REGION,), jnp.int32),
    ],
)
def _k2b(score_hbm, hists_hbm, keyc_hbm, idxc_hbm, hist2d, tot, chunk,
         keyloc, idxloc):
    wid = lax.axis_index("s") * 2 + lax.axis_index("c")
    base = wid * CHUNK
    iota = _iota16()
    zeros = jnp.zeros((16,), jnp.int32)

    pltpu.sync_copy(hists_hbm, hist2d)

    def tbody(v, _):
        acc = jnp.zeros((16,), jnp.int32)
        for t in range(NW):
            acc = acc + hist2d[t, pl.ds(v * 16, 16)]
        tot[pl.ds(v * 16, 16)] = acc
        return 0
    lax.fori_loop(0, NBINS // 16, tbody, 0)

    def sbody(v, carry):
        s, bstar = carry
        h = tot[pl.ds(v * 16, 16)]
        c = plsc.cumsum(h) + s
        bstar = bstar + jnp.sum((c <= (N - K)).astype(jnp.int32))
        return (s + jnp.sum(h), bstar)
    _, bstar = lax.fori_loop(0, NBINS // 16, sbody,
                             (jnp.int32(0), jnp.int32(0)))
    thr = lax.shift_left(bstar, 19)

    @pl.when(wid < NW - 1)
    def _():
        pltpu.sync_copy(score_hbm.at[pl.ds(base, CHUNK)], chunk)

    @pl.when(wid == NW - 1)
    def _():
        pltpu.sync_copy(score_hbm.at[pl.ds(base, CHUNK_LAST)],
                        chunk.at[pl.ds(0, CHUNK_LAST)])

    def z2body(i, _):
        keyloc[pl.ds(i * 16, 16)] = zeros
        idxloc[pl.ds(i * 16, 16)] = zeros
        return 0
    lax.fori_loop(0, REGION // 16, z2body, 0)

    def cbody(i, off):
        key = lax.bitcast_convert_type(chunk[pl.ds(i * 16, 16)], jnp.int32)
        gidx = base + i * 16 + iota
        sel = (key >= thr) & (gidx < N)
        inc = plsc.cumsum(sel.astype(jnp.int32))
        pos = off + inc - 1
        sel = sel & (pos < REGION)
        plsc.store_scatter(keyloc, [pos], key, mask=sel)
        plsc.store_scatter(idxloc, [pos], gidx, mask=sel)
        return off + jnp.sum(sel.astype(jnp.int32))
    lax.fori_loop(0, NVREG, cbody, jnp.int32(0))

    pltpu.sync_copy(keyloc, keyc_hbm.at[pl.ds(wid * REGION, REGION)])
    pltpu.sync_copy(idxloc, idxc_hbm.at[pl.ds(wid * REGION, REGION)])


def _roll(x, shift, axis):
    n = x.shape[axis]
    return pltpu.roll(x, shift % n, axis)


def _sort_body(key_ref, idx_ref, score_ref, oidx_ref):
    a = ~key_ref[...]
    b = idx_ref[...]
    iota_r = lax.broadcasted_iota(jnp.int32, (SR, SC_), 0)
    iota_c = lax.broadcasted_iota(jnp.int32, (SR, SC_), 1)
    for s in range(1, 16):
        k_ = 1 << s
        if k_ >= SC_:
            asc = (iota_r & (k_ // SC_)) == 0
        else:
            asc = (iota_c & k_) == 0
        for j in range(s - 1, -1, -1):
            d = 1 << j
            if d < SC_:
                axis, dist = 1, d
                is_lo = (iota_c & d) == 0
            else:
                axis, dist = 0, d // SC_
                is_lo = (iota_r & (d // SC_)) == 0
            pa = jnp.where(is_lo, _roll(a, -dist, axis), _roll(a, dist, axis))
            pb = jnp.where(is_lo, _roll(b, -dist, axis), _roll(b, dist, axis))
            less = (a < pa) | ((a == pa) & (b < pb))
            take_self = (asc == is_lo) == less
            a = jnp.where(take_self, a, pa)
            b = jnp.where(take_self, b, pb)
    score_ref[...] = pltpu.bitcast(~a, jnp.float32)
    oidx_ref[...] = b


def _sort32k(keyc, idxc):
    return pl.pallas_call(
        _sort_body,
        out_shape=(jax.ShapeDtypeStruct((SR, SC_), jnp.float32),
                   jax.ShapeDtypeStruct((SR, SC_), jnp.int32)),
    )(keyc.reshape(SR, SC_), idxc.reshape(SR, SC_))


GCHUNK = K // NW + 3 - 3
GROWS0 = 784
GROWS_LAST = K - 31 * GROWS0
GSUB = 112


def _scale_chunk(rows, sv, lo, cnt):
    iota = _iota16()

    def rbody(r, _):
        r0 = lo + r * 2
        for rr in range(2):
            s16 = plsc.load_gather(sv, [iota * 0 + (r0 + rr)])
            for cc in range(8):
                v = rows[r0 + rr, pl.ds(cc * 16, 16)]
                rows[r0 + rr, pl.ds(cc * 16, 16)] = v * s16
        return 0
    lax.fori_loop(0, cnt // 2, rbody, 0)


def _gather_scale(x_hbm, idx_hbm, s_hbm, out_hbm, idxv, sv, rows, sem,
                  sem_out, base, nrows):
    nch = nrows // GSUB
    rem = nrows - nch * GSUB
    pltpu.sync_copy(s_hbm.at[pl.ds(base, nrows)], sv.at[pl.ds(0, nrows)])
    copies = []
    for j in range(nch):
        pltpu.sync_copy(idx_hbm.at[pl.ds(base + j * GSUB, GSUB)], idxv.at[j])
        copies.append(pltpu.async_copy(
            x_hbm.at[idxv.at[j]], rows.at[pl.ds(j * GSUB, GSUB)], sem))
    if rem:
        pltpu.sync_copy(idx_hbm.at[pl.ds(base + nch * GSUB, rem)],
                        idxv.at[nch, pl.ds(0, rem)])
        copies.append(pltpu.async_copy(
            x_hbm.at[idxv.at[nch, pl.ds(0, rem)]],
            rows.at[pl.ds(nch * GSUB, rem)], sem))

    outs = []
    for j, cp in enumerate(copies):
        cnt = GSUB if (j < nch) else rem
        cp.wait()
        _scale_chunk(rows, sv, j * GSUB, cnt)
        outs.append(pltpu.async_copy(
            rows.at[pl.ds(j * GSUB, cnt)],
            out_hbm.at[pl.ds(base + j * GSUB, cnt)], sem_out))
    for oc in outs:
        oc.wait()


@functools.partial(
    pl.kernel,
    mesh=_MESH,
    compiler_params=pltpu.CompilerParams(needs_layout_passes=False),
    out_type=jax.ShapeDtypeStruct((K, D), jnp.float32),
    scratch_types=[
        pltpu.VMEM((GROWS0 // GSUB, GSUB), jnp.int32),
        pltpu.VMEM((GROWS0,), jnp.float32),
        pltpu.VMEM((GROWS0, D), jnp.float32),
        pltpu.SemaphoreType.DMA,
        pltpu.SemaphoreType.DMA,
    ],
)
def _k4(x_hbm, idx_hbm, s_hbm, out_hbm, idxv, sv, rows, sem, sem_out):
    wid = lax.axis_index("s") * 2 + lax.axis_index("c")
    base = wid * GROWS0

    @pl.when(wid < NW - 1)
    def _():
        _gather_scale(x_hbm, idx_hbm, s_hbm, out_hbm, idxv, sv, rows, sem,
                      sem_out, base, GROWS0)

    @pl.when(wid == NW - 1)
    def _():
        _gather_scale(x_hbm, idx_hbm, s_hbm, out_hbm, idxv, sv, rows, sem,
                      sem_out, base, GROWS_LAST)


def kernel(x, W, b):
    score = _scores(x, W, b)
    hists = _k2a(score)
    keyc, idxc = _k2b(score, hists)
    s_sorted, i_sorted = _sort32k(keyc, idxc)
    sel_s = s_sorted.reshape(M)
    idx_full = i_sorted.reshape(M)
    new_val = _k4(x, idx_full, sel_s)
    return new_val, idx_full[:K]


# speedup vs baseline: 1.0221x; 1.0221x over previous
import functools

import jax
import jax.numpy as jnp
from jax import lax
from jax.experimental import pallas as pl
from jax.experimental.pallas import tpu as pltpu
from jax.experimental.pallas import tpu_sc as plsc

N = 100000
D = 128
K = N // 4
ROWS_PER_BLK = 16384
NBLK = (N + ROWS_PER_BLK - 1) // ROWS_PER_BLK
N_PAD = NBLK * ROWS_PER_BLK

SR, SC_ = 256, 128
M = SR * SC_

BIN_SHIFT = 19
NBINS = 2048

NW = 32
CHUNK = 3136
CHUNK_LAST = N - 31 * CHUNK
NVREG = CHUNK // 16
REGION = M // NW

GROWS = K // NW + 3

_MESH = plsc.VectorSubcoreMesh(core_axis_name="c", subcore_axis_name="s")


def _iota16():
    return lax.iota(jnp.int32, 16)


def _score_body(x_ref, w_ref, b_ref, s_ref):
    zt = lax.dot_general(w_ref[...], x_ref[...],
                         (((1,), (1,)), ((), ())))
    s_ref[...] = jax.nn.sigmoid(zt + b_ref[0, 0])[0, :]


def _scores(x, W, b):
    return pl.pallas_call(
        _score_body,
        grid=(NBLK,),
        in_specs=[
            pl.BlockSpec((ROWS_PER_BLK, D), lambda i: (i, 0)),
            pl.BlockSpec((1, D), lambda i: (0, 0)),
            pl.BlockSpec((1, 1), lambda i: (0, 0)),
        ],
        out_specs=pl.BlockSpec((ROWS_PER_BLK,), lambda i: (i,)),
        out_shape=jax.ShapeDtypeStruct((N_PAD,), jnp.float32),
    )(x, W.reshape(1, D), b.reshape(1, 1))


@functools.partial(
    pl.kernel,
    mesh=_MESH,
    compiler_params=pltpu.CompilerParams(needs_layout_passes=False),
    out_type=jax.ShapeDtypeStruct((NW, NBINS), jnp.int32),
    scratch_types=[
        pltpu.VMEM((CHUNK,), jnp.float32),
        pltpu.VMEM((NBINS * 16,), jnp.int32),
        pltpu.VMEM((NBINS,), jnp.int32),
    ],
)
def _k2a(score_hbm, hists_hbm, chunk, hlanes, hloc):
    wid = lax.axis_index("s") * 2 + lax.axis_index("c")
    base = wid * CHUNK
    iota = _iota16()
    zeros = jnp.zeros((16,), jnp.int32)
    ones = jnp.ones((16,), jnp.int32)

    def zbody(i, _):
        hlanes[pl.ds(i * 16, 16)] = zeros
        return 0
    lax.fori_loop(0, NBINS, zbody, 0)

    @pl.when(wid < NW - 1)
    def _():
        pltpu.sync_copy(score_hbm.at[pl.ds(base, CHUNK)], chunk)

    @pl.when(wid == NW - 1)
    def _():
        pltpu.sync_copy(score_hbm.at[pl.ds(base, CHUNK_LAST)],
                        chunk.at[pl.ds(0, CHUNK_LAST)])

    def hbody(i, _):
        key = lax.bitcast_convert_type(chunk[pl.ds(i * 16, 16)], jnp.int32)
        b_ = lax.shift_right_logical(key, 19)
        valid = (base + i * 16 + iota) < N
        plsc.addupdate_scatter(hlanes, [b_ * 16 + iota], ones, mask=valid)
        return 0
    lax.fori_loop(0, NVREG, hbody, 0)

    def fbody(v, _):
        acc = jnp.zeros((16,), jnp.int32)
        for l in range(16):
            acc = acc + plsc.load_gather(hlanes, [v * 256 + iota * 16 + l])
        hloc[pl.ds(v * 16, 16)] = acc
        return 0
    lax.fori_loop(0, NBINS // 16, fbody, 0)

    pltpu.sync_copy(hloc, hists_hbm.at[wid])


@functools.partial(
    pl.kernel,
    mesh=_MESH,
    compiler_params=pltpu.CompilerParams(needs_layout_passes=False),
    out_type=(jax.ShapeDtypeStruct((M,), jnp.int32),
              jax.ShapeDtypeStruct((M,), jnp.int32)),
    scratch_types=[
        pltpu.VMEM((NW, NBINS), jnp.int32),
        pltpu.VMEM((NBINS,), jnp.int32),
        pltpu.VMEM((CHUNK,), jnp.float32),
        pltpu.VMEM((REGION,), jnp.int32),
        pltpu.VMEM((REGION,), jnp.int32),
    ],
)
def _k2b(score_hbm, hists_hbm, keyc_hbm, idxc_hbm, hist2d, tot, chunk,
         keyloc, idxloc):
    wid = lax.axis_index("s") * 2 + lax.axis_index("c")
    base = wid * CHUNK
    iota = _iota16()
    zeros = jnp.zeros((16,), jnp.int32)

    pltpu.sync_copy(hists_hbm, hist2d)

    def tbody(v, _):
        acc = jnp.zeros((16,), jnp.int32)
        for t in range(NW):
            acc = acc + hist2d[t, pl.ds(v * 16, 16)]
        tot[pl.ds(v * 16, 16)] = acc
        return 0
    lax.fori_loop(0, NBINS // 16, tbody, 0)

    def sbody(v, carry):
        s, bstar = carry
        h = tot[pl.ds(v * 16, 16)]
        c = plsc.cumsum(h) + s
        bstar = bstar + jnp.sum((c <= (N - K)).astype(jnp.int32))
        return (s + jnp.sum(h), bstar)
    _, bstar = lax.fori_loop(0, NBINS // 16, sbody,
                             (jnp.int32(0), jnp.int32(0)))
    thr = lax.shift_left(bstar, 19)

    @pl.when(wid < NW - 1)
    def _():
        pltpu.sync_copy(score_hbm.at[pl.ds(base, CHUNK)], chunk)

    @pl.when(wid == NW - 1)
    def _():
        pltpu.sync_copy(score_hbm.at[pl.ds(base, CHUNK_LAST)],
                        chunk.at[pl.ds(0, CHUNK_LAST)])

    def z2body(i, _):
        keyloc[pl.ds(i * 16, 16)] = zeros
        idxloc[pl.ds(i * 16, 16)] = zeros
        return 0
    lax.fori_loop(0, REGION // 16, z2body, 0)

    def cbody(i, off):
        key = lax.bitcast_convert_type(chunk[pl.ds(i * 16, 16)], jnp.int32)
        gidx = base + i * 16 + iota
        sel = (key >= thr) & (gidx < N)
        inc = plsc.cumsum(sel.astype(jnp.int32))
        pos = off + inc - 1
        sel = sel & (pos < REGION)
        plsc.store_scatter(keyloc, [pos], key, mask=sel)
        plsc.store_scatter(idxloc, [pos], gidx, mask=sel)
        return off + jnp.sum(sel.astype(jnp.int32))
    lax.fori_loop(0, NVREG, cbody, jnp.int32(0))

    pltpu.sync_copy(keyloc, keyc_hbm.at[pl.ds(wid * REGION, REGION)])
    pltpu.sync_copy(idxloc, idxc_hbm.at[pl.ds(wid * REGION, REGION)])


def _roll(x, shift, axis):
    n = x.shape[axis]
    return pltpu.roll(x, shift % n, axis)


def _sort_body(key_ref, idx_ref, score_ref, oidx_ref):
    a = ~key_ref[...]
    b = idx_ref[...]
    iota_r = lax.broadcasted_iota(jnp.int32, (SR, SC_), 0)
    iota_c = lax.broadcasted_iota(jnp.int32, (SR, SC_), 1)
    for s in range(1, 16):
        k_ = 1 << s
        if k_ >= SC_:
            asc = (iota_r & (k_ // SC_)) == 0
        else:
            asc = (iota_c & k_) == 0
        for j in range(s - 1, -1, -1):
            d = 1 << j
            if d < SC_:
                axis, dist = 1, d
                is_lo = (iota_c & d) == 0
            else:
                axis, dist = 0, d // SC_
                is_lo = (iota_r & (d // SC_)) == 0
            pa = jnp.where(is_lo, _roll(a, -dist, axis), _roll(a, dist, axis))
            pb = jnp.where(is_lo, _roll(b, -dist, axis), _roll(b, dist, axis))
            less = (a < pa) | ((a == pa) & (b < pb))
            take_self = (asc == is_lo) == less
            a = jnp.where(take_self, a, pa)
            b = jnp.where(take_self, b, pb)
    score_ref[...] = pltpu.bitcast(~a, jnp.float32)
    oidx_ref[...] = b


def _sort32k(keyc, idxc):
    return pl.pallas_call(
        _sort_body,
        out_shape=(jax.ShapeDtypeStruct((SR, SC_), jnp.float32),
                   jax.ShapeDtypeStruct((SR, SC_), jnp.int32)),
    )(keyc.reshape(SR, SC_), idxc.reshape(SR, SC_))


GCHUNK = K // NW + 3 - 3
GROWS0 = 784
GROWS_LAST = K - 31 * GROWS0
GSUB = 112


def _scale_chunk(rows, sv, lo, cnt):
    iota = _iota16()

    def rbody(r, _):
        r0 = lo + r * 2
        for rr in range(2):
            s16 = plsc.load_gather(sv, [iota * 0 + (r0 + rr)])
            for cc in range(8):
                v = rows[r0 + rr, pl.ds(cc * 16, 16)]
                rows[r0 + rr, pl.ds(cc * 16, 16)] = v * s16
        return 0
    lax.fori_loop(0, cnt // 2, rbody, 0)


def _gather_scale(x_hbm, idx_hbm, s_hbm, out_hbm, idxv, sv, rows, sem,
                  sem_out, base, nrows):
    nch = nrows // GSUB
    rem = nrows - nch * GSUB
    pltpu.sync_copy(s_hbm.at[pl.ds(base, nrows)], sv.at[pl.ds(0, nrows)])
    copies = []
    for j in range(nch):
        pltpu.sync_copy(idx_hbm.at[pl.ds(base + j * GSUB, GSUB)], idxv.at[j])
        copies.append(pltpu.async_copy(
            x_hbm.at[idxv.at[j]], rows.at[pl.ds(j * GSUB, GSUB)], sem))
    if rem:
        pltpu.sync_copy(idx_hbm.at[pl.ds(base + nch * GSUB, rem)],
                        idxv.at[nch, pl.ds(0, rem)])
        copies.append(pltpu.async_copy(
            x_hbm.at[idxv.at[nch, pl.ds(0, rem)]],
            rows.at[pl.ds(nch * GSUB, rem)], sem))

    outs = []
    for j, cp in enumerate(copies):
        cnt = GSUB if (j < nch) else rem
        cp.wait()
        _scale_chunk(rows, sv, j * GSUB, cnt)
        outs.append(pltpu.async_copy(
            rows.at[pl.ds(j * GSUB, cnt)],
            out_hbm.at[pl.ds(base + j * GSUB, cnt)], sem_out))
    for oc in outs:
        oc.wait()


@functools.partial(
    pl.kernel,
    mesh=_MESH,
    compiler_params=pltpu.CompilerParams(needs_layout_passes=False),
    out_type=jax.ShapeDtypeStruct((K, D), jnp.float32),
    scratch_types=[
        pltpu.VMEM((GROWS0 // GSUB, GSUB), jnp.int32),
        pltpu.VMEM((GROWS0,), jnp.float32),
        pltpu.VMEM((GROWS0, D), jnp.float32),
        pltpu.SemaphoreType.DMA,
        pltpu.SemaphoreType.DMA,
    ],
)
def _k4(x_hbm, idx_hbm, s_hbm, out_hbm, idxv, sv, rows, sem, sem_out):
    wid = lax.axis_index("s") * 2 + lax.axis_index("c")
    base = wid * GROWS0

    @pl.when(wid < NW - 1)
    def _():
        _gather_scale(x_hbm, idx_hbm, s_hbm, out_hbm, idxv, sv, rows, sem,
                      sem_out, base, GROWS0)

    @pl.when(wid == NW - 1)
    def _():
        _gather_scale(x_hbm, idx_hbm, s_hbm, out_hbm, idxv, sv, rows, sem,
                      sem_out, base, GROWS_LAST)


def kernel(x, W, b):
    score = _scores(x, W, b)
    hists = _k2a(score)
    keyc, idxc = _k2b(score, hists)
    s_sorted, i_sorted = _sort32k(keyc, idxc)
    sel_s = s_sorted.reshape(M)
    idx_full = i_sorted.reshape(M)
    new_val = _k4(x, idx_full, sel_s)
    return new_val, idx_full[:K]


# speedup vs baseline: 1.0240x; 1.0019x over previous
import functools

import jax
import jax.numpy as jnp
from jax import lax
from jax.experimental import pallas as pl
from jax.experimental.pallas import tpu as pltpu
from jax.experimental.pallas import tpu_sc as plsc

N = 100000
D = 128
K = N // 4
ROWS_PER_BLK = 16384
NBLK = (N + ROWS_PER_BLK - 1) // ROWS_PER_BLK
N_PAD = NBLK * ROWS_PER_BLK

SR, SC_ = 256, 128
M = SR * SC_

BIN_SHIFT = 19
NBINS = 2048

NW = 32
CHUNK = 3136
CHUNK_LAST = N - 31 * CHUNK
NVREG = CHUNK // 16
REGION = M // NW

_MESH = plsc.VectorSubcoreMesh(core_axis_name="c", subcore_axis_name="s")


def _iota16():
    return lax.iota(jnp.int32, 16)


def _score_body(x_ref, w_ref, b_ref, s_ref):
    zt = lax.dot_general(w_ref[...], x_ref[...],
                         (((1,), (1,)), ((), ())))
    s_ref[...] = jax.nn.sigmoid(zt + b_ref[0, 0])[0, :]


def _scores(x, W, b):
    return pl.pallas_call(
        _score_body,
        grid=(NBLK,),
        in_specs=[
            pl.BlockSpec((ROWS_PER_BLK, D), lambda i: (i, 0)),
            pl.BlockSpec((1, D), lambda i: (0, 0)),
            pl.BlockSpec((1, 1), lambda i: (0, 0)),
        ],
        out_specs=pl.BlockSpec((ROWS_PER_BLK,), lambda i: (i,)),
        out_shape=jax.ShapeDtypeStruct((N_PAD,), jnp.float32),
    )(x, W.reshape(1, D), b.reshape(1, 1))


@functools.partial(
    pl.kernel,
    mesh=_MESH,
    compiler_params=pltpu.CompilerParams(needs_layout_passes=False),
    out_type=jax.ShapeDtypeStruct((NW, NBINS), jnp.int32),
    scratch_types=[
        pltpu.VMEM((CHUNK,), jnp.float32),
        pltpu.VMEM((NBINS * 16,), jnp.int32),
        pltpu.VMEM((NBINS,), jnp.int32),
    ],
)
def _k2a(score_hbm, hists_hbm, chunk, hlanes, hloc):
    wid = lax.axis_index("s") * 2 + lax.axis_index("c")
    base = wid * CHUNK
    iota = _iota16()
    zeros = jnp.zeros((16,), jnp.int32)
    ones = jnp.ones((16,), jnp.int32)

    def zbody(i, _):
        hlanes[pl.ds(i * 16, 16)] = zeros
        return 0
    lax.fori_loop(0, NBINS, zbody, 0)

    @pl.when(wid < NW - 1)
    def _():
        pltpu.sync_copy(score_hbm.at[pl.ds(base, CHUNK)], chunk)

    @pl.when(wid == NW - 1)
    def _():
        pltpu.sync_copy(score_hbm.at[pl.ds(base, CHUNK_LAST)],
                        chunk.at[pl.ds(0, CHUNK_LAST)])

    def hbody(i, _):
        key = lax.bitcast_convert_type(chunk[pl.ds(i * 16, 16)], jnp.int32)
        b_ = lax.shift_right_logical(key, 19)
        valid = (base + i * 16 + iota) < N
        plsc.addupdate_scatter(hlanes, [b_ * 16 + iota], ones, mask=valid)
        return 0
    lax.fori_loop(0, NVREG, hbody, 0)

    def fbody(v, _):
        acc = jnp.zeros((16,), jnp.int32)
        for l in range(16):
            acc = acc + plsc.load_gather(hlanes, [v * 256 + iota * 16 + l])
        hloc[pl.ds(v * 16, 16)] = acc
        return 0
    lax.fori_loop(0, NBINS // 16, fbody, 0)

    pltpu.sync_copy(hloc, hists_hbm.at[wid])


@functools.partial(
    pl.kernel,
    mesh=_MESH,
    compiler_params=pltpu.CompilerParams(needs_layout_passes=False),
    out_type=(jax.ShapeDtypeStruct((M,), jnp.int32),
              jax.ShapeDtypeStruct((M,), jnp.int32)),
    scratch_types=[
        pltpu.VMEM((NW, NBINS), jnp.int32),
        pltpu.VMEM((NBINS,), jnp.int32),
        pltpu.VMEM((CHUNK,), jnp.float32),
        pltpu.VMEM((REGION,), jnp.int32),
        pltpu.VMEM((REGION,), jnp.int32),
    ],
)
def _k2b(score_hbm, hists_hbm, keyc_hbm, idxc_hbm, hist2d, tot, chunk,
         keyloc, idxloc):
    wid = lax.axis_index("s") * 2 + lax.axis_index("c")
    base = wid * CHUNK
    iota = _iota16()
    zeros = jnp.zeros((16,), jnp.int32)

    pltpu.sync_copy(hists_hbm, hist2d)

    def tbody(v, _):
        acc = jnp.zeros((16,), jnp.int32)
        for t in range(NW):
            acc = acc + hist2d[t, pl.ds(v * 16, 16)]
        tot[pl.ds(v * 16, 16)] = acc
        return 0
    lax.fori_loop(0, NBINS // 16, tbody, 0)

    def sbody(v, carry):
        s, bstar = carry
        h = tot[pl.ds(v * 16, 16)]
        c = plsc.cumsum(h) + s
        bstar = bstar + jnp.sum((c <= (N - K)).astype(jnp.int32))
        return (s + jnp.sum(h), bstar)
    _, bstar = lax.fori_loop(0, NBINS // 16, sbody,
                             (jnp.int32(0), jnp.int32(0)))
    thr = lax.shift_left(bstar, 19)

    @pl.when(wid < NW - 1)
    def _():
        pltpu.sync_copy(score_hbm.at[pl.ds(base, CHUNK)], chunk)

    @pl.when(wid == NW - 1)
    def _():
        pltpu.sync_copy(score_hbm.at[pl.ds(base, CHUNK_LAST)],
                        chunk.at[pl.ds(0, CHUNK_LAST)])

    def z2body(i, _):
        keyloc[pl.ds(i * 16, 16)] = zeros
        idxloc[pl.ds(i * 16, 16)] = zeros
        return 0
    lax.fori_loop(0, REGION // 16, z2body, 0)

    def cbody(i, off):
        key = lax.bitcast_convert_type(chunk[pl.ds(i * 16, 16)], jnp.int32)
        gidx = base + i * 16 + iota
        sel = (key >= thr) & (gidx < N)
        inc = plsc.cumsum(sel.astype(jnp.int32))
        pos = off + inc - 1
        sel = sel & (pos < REGION)
        plsc.store_scatter(keyloc, [pos], key, mask=sel)
        plsc.store_scatter(idxloc, [pos], gidx, mask=sel)
        return off + jnp.sum(sel.astype(jnp.int32))
    lax.fori_loop(0, NVREG, cbody, jnp.int32(0))

    pltpu.sync_copy(keyloc, keyc_hbm.at[pl.ds(wid * REGION, REGION)])
    pltpu.sync_copy(idxloc, idxc_hbm.at[pl.ds(wid * REGION, REGION)])


def _roll(x, shift, axis):
    n = x.shape[axis]
    return pltpu.roll(x, shift % n, axis)


def _sort_body(key_ref, idx_ref, score_ref, oidx_ref):
    a = ~key_ref[...]
    b = idx_ref[...]
    iota_r = lax.broadcasted_iota(jnp.int32, (SR, SC_), 0)
    iota_c = lax.broadcasted_iota(jnp.int32, (SR, SC_), 1)
    for s in range(1, 16):
        k_ = 1 << s
        if k_ >= SC_:
            asc = (iota_r & (k_ // SC_)) == 0
        else:
            asc = (iota_c & k_) == 0
        for j in range(s - 1, -1, -1):
            d = 1 << j
            if d < SC_:
                axis, dist = 1, d
                is_lo = (iota_c & d) == 0
            else:
                axis, dist = 0, d // SC_
                is_lo = (iota_r & (d // SC_)) == 0
            pa = jnp.where(is_lo, _roll(a, -dist, axis), _roll(a, dist, axis))
            pb = jnp.where(is_lo, _roll(b, -dist, axis), _roll(b, dist, axis))
            less = (a < pa) | ((a == pa) & (b < pb))
            take_self = (asc == is_lo) == less
            a = jnp.where(take_self, a, pa)
            b = jnp.where(take_self, b, pb)
    score_ref[...] = pltpu.bitcast(~a, jnp.float32)
    oidx_ref[...] = b


def _sort32k(keyc, idxc):
    return pl.pallas_call(
        _sort_body,
        out_shape=(jax.ShapeDtypeStruct((SR, SC_), jnp.float32),
                   jax.ShapeDtypeStruct((SR, SC_), jnp.int32)),
    )(keyc.reshape(SR, SC_), idxc.reshape(SR, SC_))


GROWS0 = 784
GROWS_LAST = K - 31 * GROWS0
GSUB = 112


def _scale_chunk(rows, sv, lo, cnt):
    iota = _iota16()

    def rbody(r, _):
        r0 = lo + r * 2
        for rr in range(2):
            s16 = plsc.load_gather(sv, [iota * 0 + (r0 + rr)])
            for cc in range(8):
                v = rows[r0 + rr, pl.ds(cc * 16, 16)]
                rows[r0 + rr, pl.ds(cc * 16, 16)] = v * s16
        return 0
    lax.fori_loop(0, cnt // 2, rbody, 0)


def _gather_scale(x_hbm, idx_hbm, s_hbm, out_hbm, idxv, sv, rows, sem,
                  sem_out, base, nrows):
    nch = nrows // GSUB
    rem = nrows - nch * GSUB
    pltpu.sync_copy(s_hbm.at[pl.ds(base, nrows)], sv.at[pl.ds(0, nrows)])
    copies = []
    for j in range(nch):
        pltpu.sync_copy(idx_hbm.at[pl.ds(base + j * GSUB, GSUB)], idxv.at[j])
        copies.append(pltpu.async_copy(
            x_hbm.at[idxv.at[j]], rows.at[pl.ds(j * GSUB, GSUB)], sem))
    if rem:
        pltpu.sync_copy(idx_hbm.at[pl.ds(base + nch * GSUB, rem)],
                        idxv.at[nch, pl.ds(0, rem)])
        copies.append(pltpu.async_copy(
            x_hbm.at[idxv.at[nch, pl.ds(0, rem)]],
            rows.at[pl.ds(nch * GSUB, rem)], sem))

    outs = []
    for j, cp in enumerate(copies):
        cnt = GSUB if (j < nch) else rem
        cp.wait()
        _scale_chunk(rows, sv, j * GSUB, cnt)
        outs.append(pltpu.async_copy(
            rows.at[pl.ds(j * GSUB, cnt)],
            out_hbm.at[pl.ds(base + j * GSUB, cnt)], sem_out))
    for oc in outs:
        oc.wait()


@functools.partial(
    pl.kernel,
    mesh=_MESH,
    compiler_params=pltpu.CompilerParams(needs_layout_passes=False),
    out_type=jax.ShapeDtypeStruct((K, D), jnp.float32),
    scratch_types=[
        pltpu.VMEM((GROWS0 // GSUB, GSUB), jnp.int32),
        pltpu.VMEM((GROWS0,), jnp.float32),
        pltpu.VMEM((GROWS0, D), jnp.float32),
        pltpu.SemaphoreType.DMA,
        pltpu.SemaphoreType.DMA,
    ],
)
def _k4(x_hbm, idx_hbm, s_hbm, out_hbm, idxv, sv, rows, sem, sem_out):
    wid = lax.axis_index("s") * 2 + lax.axis_index("c")
    base = wid * GROWS0

    @pl.when(wid < NW - 1)
    def _():
        _gather_scale(x_hbm, idx_hbm, s_hbm, out_hbm, idxv, sv, rows, sem,
                      sem_out, base, GROWS0)

    @pl.when(wid == NW - 1)
    def _():
        _gather_scale(x_hbm, idx_hbm, s_hbm, out_hbm, idxv, sv, rows, sem,
                      sem_out, base, GROWS_LAST)


def kernel(x, W, b):
    score = _scores(x, W, b)
    hists = _k2a(score)
    keyc, idxc = _k2b(score, hists)
    s_sorted, i_sorted = _sort32k(keyc, idxc)
    sel_s = s_sorted.reshape(M)
    idx_full = i_sorted.reshape(M)
    new_val = _k4(x, idx_full, sel_s)
    return new_val, idx_full[:K]
